# baseline (speedup 1.0000x reference)
import functools

import jax
import jax.numpy as jnp
from jax import lax
from jax.experimental import pallas as pl
from jax.experimental.pallas import tpu as pltpu
from jax.experimental.pallas import tpu_sc as plsc

NTOKEN = 100000
DMODEL = 1024
SEQLEN = 2048
BATCH = 4
EPS = 1e-5

NC = 2
NS = 16
NW = NC * NS
NTOK = BATCH * SEQLEN
TOK_PER_W = NTOK // NW
CH = 32
NCH = TOK_PER_W // CH
LANES = 16
NVREG = DMODEL // LANES


def _lane_total(x):
    iota = lax.iota(jnp.int32, LANES)
    for sh in (8, 4, 2, 1):
        x = x + x.at[jnp.bitwise_xor(iota, sh)].get(mode="promise_in_bounds")
    return x


def _rsqrt(x):
    i = plsc.bitcast(x, jnp.int32)
    i = jnp.int32(0x5F3759DF) - lax.shift_right_logical(i, 1)
    y = plsc.bitcast(i, jnp.float32)
    half = x * 0.5
    for _ in range(3):
        y = y * (1.5 - half * y * y)
    return y


def _body(ids_hbm, segs_hbm, tok_tab, pos_tab, seg_tab, lnw_hbm, lnb_hbm,
          out_hbm, idx_v, seg_v, rows_v, pos_v, segtab_v, lnw_v,
          lnb_v, sem):
    wid = lax.axis_index("s") * NC + lax.axis_index("c")
    base = wid * TOK_PER_W

    pltpu.sync_copy(seg_tab, segtab_v)
    pltpu.sync_copy(lnw_hbm, lnw_v)
    pltpu.sync_copy(lnb_hbm, lnb_v)

    def chunk_body(c, _):
        tok_base = pl.multiple_of(base + c * CH, CH)
        pos_base = pl.multiple_of(
            lax.rem(base, jnp.int32(SEQLEN)) + c * CH, CH)

        pltpu.sync_copy(ids_hbm.at[pl.ds(tok_base, CH)], idx_v)
        pltpu.sync_copy(segs_hbm.at[pl.ds(tok_base, CH)], seg_v)
        gather = pltpu.async_copy(tok_tab.at[idx_v], rows_v, sem)
        pltpu.sync_copy(pos_tab.at[pl.ds(pos_base, CH)], pos_v)
        gather.wait()

        def tok_body(t, _):
            toff = plsc.load_gather(seg_v, [jnp.full((LANES,), t, jnp.int32)])
            iota = lax.iota(jnp.int32, LANES)

            def sum_body(j, carry):
                acc, acc2 = carry
                sl = pl.ds(j * LANES, LANES)
                seg = plsc.load_gather(segtab_v, [toff + (iota + j * LANES)])
                x = rows_v[t, sl] + pos_v[t, sl] + seg
                rows_v[t, sl] = x
                return acc + x, acc2 + x * x

            zero = jnp.zeros((LANES,), jnp.float32)
            acc, acc2 = lax.fori_loop(0, NVREG, sum_body, (zero, zero))
            mean = _lane_total(acc) * (1.0 / DMODEL)
            ex2 = _lane_total(acc2) * (1.0 / DMODEL)
            rstd = _rsqrt(ex2 - mean * mean + EPS)

            def norm_body(j, _):
                sl = pl.ds(j * LANES, LANES)
                x = rows_v[t, sl]
                rows_v[t, sl] = (x - mean) * rstd * lnw_v[sl] + lnb_v[sl]
                return 0

            lax.fori_loop(0, NVREG, norm_body, 0)
            return 0

        lax.fori_loop(0, CH, tok_body, 0)
        pltpu.sync_copy(rows_v, out_hbm.at[pl.ds(tok_base, CH)])
        return 0

    lax.fori_loop(0, NCH, chunk_body, 0)


@jax.jit
def kernel(input_ids, segments, token_table, position_table, segment_table,
           ln_weight, ln_bias):
    ids = input_ids.reshape(NTOK).astype(jnp.int32)
    segs = segments.reshape(NTOK).astype(jnp.int32) * DMODEL
    mesh = plsc.VectorSubcoreMesh(
        core_axis_name="c", subcore_axis_name="s",
        num_cores=NC, num_subcores=NS)
    run = pl.kernel(
        _body,
        out_type=jax.ShapeDtypeStruct((NTOK, DMODEL), jnp.float32),
        mesh=mesh,
        compiler_params=pltpu.CompilerParams(needs_layout_passes=False),
        scratch_types=[
            pltpu.VMEM((CH,), jnp.int32),
            pltpu.VMEM((CH,), jnp.int32),
            pltpu.VMEM((CH, DMODEL), jnp.float32),
            pltpu.VMEM((CH, DMODEL), jnp.float32),
            pltpu.VMEM((3 * DMODEL,), jnp.float32),
            pltpu.VMEM((DMODEL,), jnp.float32),
            pltpu.VMEM((DMODEL,), jnp.float32),
            pltpu.SemaphoreType.DMA,
        ],
    )
    out = run(ids, segs, token_table, position_table,
              segment_table.reshape(3 * DMODEL), ln_weight, ln_bias)
    return out.reshape(BATCH, SEQLEN, DMODEL)


# speedup vs baseline: 1.4667x; 1.4667x over previous
import jax
import jax.numpy as jnp
from jax import lax
from jax.experimental import pallas as pl
from jax.experimental.pallas import tpu as pltpu
from jax.experimental.pallas import tpu_sc as plsc

NTOKEN = 100000
DMODEL = 1024
SEQLEN = 2048
BATCH = 4
EPS = 1e-5

NC = 2
NS = 16
NW = NC * NS
NTOK = BATCH * SEQLEN
TOK_PER_W = NTOK // NW
CH = 16
NCH = TOK_PER_W // CH
LANES = 16
NVREG = DMODEL // LANES
UNROLL = 4


def _lane_total(x):
    iota = lax.iota(jnp.int32, LANES)
    for sh in (8, 4, 2, 1):
        x = x + x.at[jnp.bitwise_xor(iota, sh)].get(mode="promise_in_bounds")
    return x


def _rsqrt(x):
    i = plsc.bitcast(x, jnp.int32)
    i = jnp.int32(0x5F3759DF) - lax.shift_right_logical(i, 1)
    y = plsc.bitcast(i, jnp.float32)
    half = x * 0.5
    for _ in range(3):
        y = y * (1.5 - half * y * y)
    return y


def _body(ids_hbm, segs_hbm, tok_tab, pos_tab, seg_tab, lnw_hbm, lnb_hbm,
          out_hbm, ids_v, segs_v, rows0, rows1, pos0, pos1, srow0, srow1,
          lnw_v, lnb_v, sem0, sem1, osem0, osem1):
    wid = lax.axis_index("s") * NC + lax.axis_index("c")
    base = wid * TOK_PER_W
    sbase = lax.rem(base, jnp.int32(SEQLEN))

    rows = (rows0, rows1)
    pos = (pos0, pos1)
    srow = (srow0, srow1)
    sems = (sem0, sem1)
    osems = (osem0, osem1)

    pltpu.sync_copy(ids_hbm.at[pl.ds(base, TOK_PER_W)], ids_v)
    pltpu.sync_copy(segs_hbm.at[pl.ds(base, TOK_PER_W)], segs_v)
    pltpu.sync_copy(lnw_hbm, lnw_v)
    pltpu.sync_copy(lnb_hbm, lnb_v)

    def issue_in(n, p):
        off = pl.multiple_of(n * CH, CH)
        idx = ids_v[pl.ds(off, CH)]
        sdx = segs_v[pl.ds(off, CH)]
        pltpu.async_copy(tok_tab.at[idx], rows[p], sems[p])
        pltpu.async_copy(seg_tab.at[sdx], srow[p], sems[p])
        pltpu.async_copy(pos_tab.at[pl.ds(sbase + off, CH)], pos[p], sems[p])

    def wait_in(p):
        pltpu.make_async_copy(tok_tab.at[pl.ds(0, CH)], rows[p],
                              sems[p]).wait()
        pltpu.make_async_copy(pos_tab.at[pl.ds(0, CH)], srow[p],
                              sems[p]).wait()
        pltpu.make_async_copy(pos_tab.at[pl.ds(0, CH)], pos[p],
                              sems[p]).wait()

    def wait_out(p):
        pltpu.make_async_copy(rows[p], out_hbm.at[pl.ds(0, CH)],
                              osems[p]).wait()

    def compute(n, p):
        rv, pv, sv = rows[p], pos[p], srow[p]

        def tok_body(t, _):
            def sum_body(j, carry):
                acc, acc2 = carry
                sl = pl.ds(j * LANES, LANES)
                x = rv[t, sl] + pv[t, sl] + sv[t, sl]
                rv[t, sl] = x
                return acc + x, acc2 + x * x

            zero = jnp.zeros((LANES,), jnp.float32)
            acc, acc2 = lax.fori_loop(0, NVREG, sum_body, (zero, zero),
                                      unroll=UNROLL)
            mean = _lane_total(acc) * (1.0 / DMODEL)
            ex2 = _lane_total(acc2) * (1.0 / DMODEL)
            rstd = _rsqrt(ex2 - mean * mean + EPS)
            shift = mean * rstd

            def norm_body(j, _):
                sl = pl.ds(j * LANES, LANES)
                x = rv[t, sl]
                rv[t, sl] = (x * rstd - shift) * lnw_v[sl] + lnb_v[sl]
                return 0

            lax.fori_loop(0, NVREG, norm_body, 0, unroll=UNROLL)
            return 0

        lax.fori_loop(0, CH, tok_body, 0)

    issue_in(jnp.int32(0), 0)
    issue_in(jnp.int32(1), 1)

    def step(g, _):
        for p in (0, 1):
            n = 2 * g + p
            wait_in(p)
            compute(n, p)
            off = pl.multiple_of(base + n * CH, CH)
            pltpu.async_copy(rows[p], out_hbm.at[pl.ds(off, CH)], osems[p])

            @pl.when(n + 2 < NCH)
            def _():
                wait_out(p)
                issue_in(n + 2, p)

        return 0

    lax.fori_loop(0, NCH // 2, step, 0)
    wait_out(0)
    wait_out(1)


@jax.jit
def kernel(input_ids, segments, token_table, position_table, segment_table,
           ln_weight, ln_bias):
    ids = input_ids.reshape(NTOK).astype(jnp.int32)
    segs = segments.reshape(NTOK).astype(jnp.int32)
    mesh = plsc.VectorSubcoreMesh(
        core_axis_name="c", subcore_axis_name="s",
        num_cores=NC, num_subcores=NS)
    run = pl.kernel(
        _body,
        out_type=jax.ShapeDtypeStruct((NTOK, DMODEL), jnp.float32),
        mesh=mesh,
        compiler_params=pltpu.CompilerParams(needs_layout_passes=False),
        scratch_types=[
            pltpu.VMEM((TOK_PER_W,), jnp.int32),
            pltpu.VMEM((TOK_PER_W,), jnp.int32),
            pltpu.VMEM((CH, DMODEL), jnp.float32),
            pltpu.VMEM((CH, DMODEL), jnp.float32),
            pltpu.VMEM((CH, DMODEL), jnp.float32),
            pltpu.VMEM((CH, DMODEL), jnp.float32),
            pltpu.VMEM((CH, DMODEL), jnp.float32),
            pltpu.VMEM((CH, DMODEL), jnp.float32),
            pltpu.VMEM((DMODEL,), jnp.float32),
            pltpu.VMEM((DMODEL,), jnp.float32),
            pltpu.SemaphoreType.DMA,
            pltpu.SemaphoreType.DMA,
            pltpu.SemaphoreType.DMA,
            pltpu.SemaphoreType.DMA,
        ],
    )
    out = run(ids, segs, token_table, position_table, segment_table,
              ln_weight, ln_bias)
    return out.reshape(BATCH, SEQLEN, DMODEL)


# speedup vs baseline: 3.4525x; 2.3539x over previous
import functools

import jax
import jax.numpy as jnp
from jax import lax
from jax.experimental import pallas as pl
from jax.experimental.pallas import tpu as pltpu
from jax.experimental.pallas import tpu_sc as plsc

NTOKEN = 100000
DMODEL = 1024
SEQLEN = 2048
BATCH = 4
EPS = 1e-5

NC = 2
NS = 16
NW = NC * NS
NTOK = BATCH * SEQLEN
TOK_PER_W = NTOK // NW
CH = 32
NCH = TOK_PER_W // CH
NBUF = 3

BS = 256
GRID = NTOK // BS


def _sc_body(ids_hbm, tok_tab, out_hbm, ids_v, b0, b1, b2, g0, g1, g2,
             o0, o1, o2):
    wid = lax.axis_index("s") * NC + lax.axis_index("c")
    base = wid * TOK_PER_W
    bufs = (b0, b1, b2)
    gsems = (g0, g1, g2)
    osems = (o0, o1, o2)

    pltpu.sync_copy(ids_hbm.at[pl.ds(base, TOK_PER_W)], ids_v)

    def issue_gather(n, p):
        idx = ids_v.at[pl.ds(n * CH, CH)]
        pltpu.async_copy(tok_tab.at[idx], bufs[p], gsems[p])

    def finish(n, p):
        pltpu.make_async_copy(tok_tab.at[pl.ds(0, CH)], bufs[p],
                              gsems[p]).wait()
        off = pl.multiple_of(base + n * CH, CH)
        pltpu.async_copy(bufs[p], out_hbm.at[pl.ds(off, CH)], osems[p])

    for n in range(NCH):
        p = n % NBUF
        if n >= NBUF:
            pltpu.make_async_copy(bufs[p], out_hbm.at[pl.ds(0, CH)],
                                  osems[p]).wait()
        issue_gather(n, p)
        if n >= 1:
            finish(n - 1, (n - 1) % NBUF)
    finish(NCH - 1, (NCH - 1) % NBUF)
    for p in range(NBUF - 1, -1, -1):
        pltpu.make_async_copy(bufs[p], out_hbm.at[pl.ds(0, CH)],
                              osems[p]).wait()


def _sc_gather(ids, token_table):
    mesh = plsc.VectorSubcoreMesh(
        core_axis_name="c", subcore_axis_name="s",
        num_cores=NC, num_subcores=NS)
    run = pl.kernel(
        _sc_body,
        out_type=jax.ShapeDtypeStruct((NTOK, DMODEL), jnp.float32),
        mesh=mesh,
        compiler_params=pltpu.CompilerParams(needs_layout_passes=False),
        scratch_types=(
            [pltpu.VMEM((TOK_PER_W,), jnp.int32)]
            + [pltpu.VMEM((CH, DMODEL), jnp.float32)] * NBUF
            + [pltpu.SemaphoreType.DMA] * (2 * NBUF)
        ),
    )
    return run(ids, token_table)


def _tc_body(x_ref, pos_ref, seg_ref, segtab_ref, lnw_ref, lnb_ref, out_ref):
    x = x_ref[...]
    seg = seg_ref[...]
    lanes = lax.broadcasted_iota(jnp.int32, (BS, 3), 1)
    onehot = (seg == lanes).astype(jnp.float32)
    seg_emb = jnp.dot(onehot, segtab_ref[...],
                      preferred_element_type=jnp.float32)
    x = x + pos_ref[...] + seg_emb
    mean = jnp.mean(x, axis=-1, keepdims=True)
    xc = x - mean
    var = jnp.mean(xc * xc, axis=-1, keepdims=True)
    y = xc * lax.rsqrt(var + EPS)
    out_ref[...] = y * lnw_ref[...] + lnb_ref[...]


def _tc_finish(x, segs, position_table, segment_table, ln_weight, ln_bias):
    return pl.pallas_call(
        _tc_body,
        grid=(GRID,),
        in_specs=[
            pl.BlockSpec((BS, DMODEL), lambda i: (i, 0)),
            pl.BlockSpec((BS, DMODEL), lambda i: (i % (SEQLEN // BS), 0)),
            pl.BlockSpec((BS, 1), lambda i: (i, 0)),
            pl.BlockSpec((3, DMODEL), lambda i: (0, 0)),
            pl.BlockSpec((1, DMODEL), lambda i: (0, 0)),
            pl.BlockSpec((1, DMODEL), lambda i: (0, 0)),
        ],
        out_specs=pl.BlockSpec((BS, DMODEL), lambda i: (i, 0)),
        out_shape=jax.ShapeDtypeStruct((NTOK, DMODEL), jnp.float32),
        compiler_params=pltpu.CompilerParams(
            dimension_semantics=("arbitrary",)),
    )(x, position_table, segs, segment_table,
      ln_weight.reshape(1, DMODEL), ln_bias.reshape(1, DMODEL))


@jax.jit
def kernel(input_ids, segments, token_table, position_table, segment_table,
           ln_weight, ln_bias):
    ids = input_ids.reshape(NTOK).astype(jnp.int32)
    segs = segments.reshape(NTOK, 1).astype(jnp.int32)
    gathered = _sc_gather(ids, token_table)
    out = _tc_finish(gathered, segs, position_table, segment_table,
                     ln_weight, ln_bias)
    return out.reshape(BATCH, SEQLEN, DMODEL)


# speedup vs baseline: 3.5545x; 1.0295x over previous
import functools

import jax
import jax.numpy as jnp
from jax import lax
from jax.experimental import pallas as pl
from jax.experimental.pallas import tpu as pltpu
from jax.experimental.pallas import tpu_sc as plsc

NTOKEN = 100000
DMODEL = 1024
SEQLEN = 2048
BATCH = 4
EPS = 1e-5

NC = 2
NS = 16
NW = NC * NS
NTOK = BATCH * SEQLEN
TOK_PER_W = NTOK // NW
CH = 32
NCH = TOK_PER_W // CH
NBUF = 3

BS = 256
GRID = NTOK // BS


def _sc_body(ids_hbm, tok_tab, out_hbm, ids_v, b0, b1, b2, g0, g1, g2,
             o0, o1, o2):
    wid = lax.axis_index("s") * NC + lax.axis_index("c")
    base = wid * TOK_PER_W
    bufs = (b0, b1, b2)
    gsems = (g0, g1, g2)
    osems = (o0, o1, o2)

    pltpu.sync_copy(ids_hbm.at[pl.ds(base, TOK_PER_W)], ids_v)

    def issue_gather(n, p):
        idx = ids_v.at[pl.ds(n * CH, CH)]
        pltpu.async_copy(tok_tab.at[idx], bufs[p], gsems[p])

    def finish(n, p):
        pltpu.make_async_copy(tok_tab.at[pl.ds(0, CH)], bufs[p],
                              gsems[p]).wait()
        off = pl.multiple_of(base + n * CH, CH)
        pltpu.async_copy(bufs[p], out_hbm.at[pl.ds(off, CH)], osems[p])

    for n in range(NCH):
        p = n % NBUF
        if n >= NBUF:
            pltpu.make_async_copy(bufs[p], out_hbm.at[pl.ds(0, CH)],
                                  osems[p]).wait()
        issue_gather(n, p)
        if n >= 1:
            finish(n - 1, (n - 1) % NBUF)
    finish(NCH - 1, (NCH - 1) % NBUF)
    for p in range(NBUF - 1, -1, -1):
        pltpu.make_async_copy(bufs[p], out_hbm.at[pl.ds(0, CH)],
                              osems[p]).wait()


def _sc_gather(ids, token_table):
    mesh = plsc.VectorSubcoreMesh(
        core_axis_name="c", subcore_axis_name="s",
        num_cores=NC, num_subcores=NS)
    run = pl.kernel(
        _sc_body,
        out_type=jax.ShapeDtypeStruct((NTOK, DMODEL), jnp.float32),
        mesh=mesh,
        compiler_params=pltpu.CompilerParams(needs_layout_passes=False),
        scratch_types=(
            [pltpu.VMEM((TOK_PER_W,), jnp.int32)]
            + [pltpu.VMEM((CH, DMODEL), jnp.float32)] * NBUF
            + [pltpu.SemaphoreType.DMA] * (2 * NBUF)
        ),
    )
    return run(ids, token_table)


def _tc_body(x_ref, pos_ref, seg_ref, segtab_ref, lnw_ref, lnb_ref, out_ref):
    x = x_ref[...]
    seg = seg_ref[...]
    lanes = lax.broadcasted_iota(jnp.int32, (BS, 3), 1)
    onehot = (seg == lanes).astype(jnp.float32)
    seg_emb = jnp.dot(onehot, segtab_ref[...],
                      preferred_element_type=jnp.float32)
    x = x + pos_ref[...] + seg_emb
    mean = jnp.mean(x, axis=-1, keepdims=True)
    xc = x - mean
    var = jnp.mean(xc * xc, axis=-1, keepdims=True)
    y = xc * lax.rsqrt(var + EPS)
    out_ref[...] = y * lnw_ref[...] + lnb_ref[...]


def _tc_finish(x, segs, position_table, segment_table, ln_weight, ln_bias):
    return pl.pallas_call(
        _tc_body,
        grid=(GRID,),
        in_specs=[
            pl.BlockSpec((BS, DMODEL),
                         lambda i: ((i % BATCH) * (SEQLEN // BS) + i // BATCH,
                                    0)),
            pl.BlockSpec((BS, DMODEL), lambda i: (i // BATCH, 0)),
            pl.BlockSpec((BS, 1),
                         lambda i: ((i % BATCH) * (SEQLEN // BS) + i // BATCH,
                                    0)),
            pl.BlockSpec((3, DMODEL), lambda i: (0, 0)),
            pl.BlockSpec((1, DMODEL), lambda i: (0, 0)),
            pl.BlockSpec((1, DMODEL), lambda i: (0, 0)),
        ],
        out_specs=pl.BlockSpec(
            (BS, DMODEL),
            lambda i: ((i % BATCH) * (SEQLEN // BS) + i // BATCH, 0)),
        out_shape=jax.ShapeDtypeStruct((NTOK, DMODEL), jnp.float32),
        compiler_params=pltpu.CompilerParams(
            dimension_semantics=("arbitrary",)),
    )(x, position_table, segs, segment_table,
      ln_weight.reshape(1, DMODEL), ln_bias.reshape(1, DMODEL))


@jax.jit
def kernel(input_ids, segments, token_table, position_table, segment_table,
           ln_weight, ln_bias):
    ids = input_ids.reshape(NTOK).astype(jnp.int32)
    segs = segments.reshape(NTOK, 1).astype(jnp.int32)
    gathered = _sc_gather(ids, token_table)
    out = _tc_finish(gathered, segs, position_table, segment_table,
                     ln_weight, ln_bias)
    return out.reshape(BATCH, SEQLEN, DMODEL)


# speedup vs baseline: 3.8790x; 1.0913x over previous
import functools

import jax
import jax.numpy as jnp
from jax import lax
from jax.experimental import pallas as pl
from jax.experimental.pallas import tpu as pltpu
from jax.experimental.pallas import tpu_sc as plsc

NTOKEN = 100000
DMODEL = 1024
SEQLEN = 2048
BATCH = 4
EPS = 1e-5

NC = 2
NS = 16
NW = NC * NS
NTOK = BATCH * SEQLEN
NSLICE = 2
NTOKK = NTOK // NSLICE
NBAT = NTOKK // SEQLEN
TOK_PER_W = NTOKK // NW
CH = 32
NCH = TOK_PER_W // CH
NBUF = 3

BS = 512
SBLK = SEQLEN // BS
GRIDK = NTOKK // BS


def _sc_body(ids_hbm, tok_tab, out_hbm, ids_v, b0, b1, b2, g0, g1, g2,
             o0, o1, o2):
    wid = lax.axis_index("s") * NC + lax.axis_index("c")
    base = wid * TOK_PER_W
    bufs = (b0, b1, b2)
    gsems = (g0, g1, g2)
    osems = (o0, o1, o2)

    pltpu.sync_copy(ids_hbm.at[pl.ds(base, TOK_PER_W)], ids_v)

    def issue_gather(n, p):
        idx = ids_v.at[pl.ds(n * CH, CH)]
        pltpu.async_copy(tok_tab.at[idx], bufs[p], gsems[p])

    def finish(n, p):
        pltpu.make_async_copy(tok_tab.at[pl.ds(0, CH)], bufs[p],
                              gsems[p]).wait()
        off = pl.multiple_of(base + n * CH, CH)
        pltpu.async_copy(bufs[p], out_hbm.at[pl.ds(off, CH)], osems[p])

    for n in range(NCH):
        p = n % NBUF
        if n >= NBUF:
            pltpu.make_async_copy(bufs[p], out_hbm.at[pl.ds(0, CH)],
                                  osems[p]).wait()
        issue_gather(n, p)
        if n >= 1:
            finish(n - 1, (n - 1) % NBUF)
    finish(NCH - 1, (NCH - 1) % NBUF)
    for p in range(NBUF - 1, -1, -1):
        pltpu.make_async_copy(bufs[p], out_hbm.at[pl.ds(0, CH)],
                              osems[p]).wait()


def _sc_gather(ids_slice, token_table):
    mesh = plsc.VectorSubcoreMesh(
        core_axis_name="c", subcore_axis_name="s",
        num_cores=NC, num_subcores=NS)
    run = pl.kernel(
        _sc_body,
        out_type=jax.ShapeDtypeStruct((NTOKK, DMODEL), jnp.float32),
        mesh=mesh,
        compiler_params=pltpu.CompilerParams(needs_layout_passes=False),
        scratch_types=(
            [pltpu.VMEM((TOK_PER_W,), jnp.int32)]
            + [pltpu.VMEM((CH, DMODEL), jnp.float32)] * NBUF
            + [pltpu.SemaphoreType.DMA] * (2 * NBUF)
        ),
    )
    return run(ids_slice, token_table)


def _tc_body(x_ref, pos_ref, seg_ref, segtab_ref, lnw_ref,
             lnb_ref, out_ref):
    seg = seg_ref[...]
    lanes = lax.broadcasted_iota(jnp.int32, (BS, 3), 1)
    onehot = (seg == lanes).astype(jnp.float32)
    seg_emb = jnp.dot(onehot, segtab_ref[...],
                      preferred_element_type=jnp.float32)
    x = (x_ref[...] + pos_ref[...]) + seg_emb
    ones = jnp.ones((DMODEL, 1), jnp.float32)
    s1 = jnp.dot(x, ones, preferred_element_type=jnp.float32)
    s2 = jnp.dot(x * x, ones, preferred_element_type=jnp.float32)
    mean = s1 * (1.0 / DMODEL)
    var = s2 * (1.0 / DMODEL) - mean * mean
    rstd = lax.rsqrt(var + EPS)
    w = lnw_ref[...]
    out_ref[...] = (x * rstd - mean * rstd) * w + lnb_ref[...]


def _tc_finish(prev, x, segs_s, position_table, segment_table, lnw, lnb, s):
    blk_off = s * NBAT * SBLK

    def xmap(i):
        return ((i % NBAT) * SBLK + i // NBAT, 0)

    def omap(i):
        return (blk_off + (i % NBAT) * SBLK + i // NBAT, 0)

    def body(*refs):
        if prev is None:
            _tc_body(*refs)
        else:
            _tc_body(*refs[1:])

    in_specs = [
        pl.BlockSpec((BS, DMODEL), xmap),
        pl.BlockSpec((BS, DMODEL), lambda i: (i // NBAT, 0)),
        pl.BlockSpec((BS, 1), xmap),
        pl.BlockSpec((3, DMODEL), lambda i: (0, 0)),
        pl.BlockSpec((1, DMODEL), lambda i: (0, 0)),
        pl.BlockSpec((1, DMODEL), lambda i: (0, 0)),
    ]
    args = [x, position_table, segs_s, segment_table,
            lnw.reshape(1, DMODEL), lnb.reshape(1, DMODEL)]
    aliases = {}
    if prev is not None:
        in_specs = [pl.BlockSpec(memory_space=pl.ANY)] + in_specs
        args = [prev] + args
        aliases = {0: 0}
    return pl.pallas_call(
        body,
        grid=(GRIDK,),
        in_specs=in_specs,
        out_specs=pl.BlockSpec((BS, DMODEL), omap),
        out_shape=jax.ShapeDtypeStruct((NTOK, DMODEL), jnp.float32),
        input_output_aliases=aliases,
        compiler_params=pltpu.CompilerParams(
            dimension_semantics=("arbitrary",)),
    )(*args)


@jax.jit
def kernel(input_ids, segments, token_table, position_table, segment_table,
           ln_weight, ln_bias):
    ids = input_ids.reshape(NTOK).astype(jnp.int32)
    segs = segments.reshape(NTOK, 1).astype(jnp.int32)
    gathered = [
        _sc_gather(ids[s * NTOKK:(s + 1) * NTOKK], token_table)
        for s in range(NSLICE)
    ]
    out = None
    for s in range(NSLICE):
        out = _tc_finish(out, gathered[s],
                         segs[s * NTOKK:(s + 1) * NTOKK],
                         position_table, segment_table,
                         ln_weight, ln_bias, s)
    return out.reshape(BATCH, SEQLEN, DMODEL)


# speedup vs baseline: 3.9012x; 1.0057x over previous
import functools

import jax
import jax.numpy as jnp
from jax import lax
from jax.experimental import pallas as pl
from jax.experimental.pallas import tpu as pltpu
from jax.experimental.pallas import tpu_sc as plsc

NTOKEN = 100000
DMODEL = 1024
SEQLEN = 2048
BATCH = 4
EPS = 1e-5

NC = 2
NS = 16
NW = NC * NS
NTOK = BATCH * SEQLEN
NSLICE = 2
NTOKK = NTOK // NSLICE
NBAT = NTOKK // SEQLEN
TOK_PER_W = NTOKK // NW
CH = 16
NCH = TOK_PER_W // CH
NBUF = 6
LOOKAHEAD = 4

BS = 512
SBLK = SEQLEN // BS
GRIDK = NTOKK // BS


def _sc_body(ids_hbm, tok_tab, out_hbm, ids_v, *scratch):
    wid = lax.axis_index("s") * NC + lax.axis_index("c")
    base = wid * TOK_PER_W
    bufs = scratch[:NBUF]
    gsems = scratch[NBUF:2 * NBUF]
    osems = scratch[2 * NBUF:3 * NBUF]

    pltpu.sync_copy(ids_hbm.at[pl.ds(base, TOK_PER_W)], ids_v)

    def issue_gather(n, p):
        idx = ids_v.at[pl.ds(n * CH, CH)]
        pltpu.async_copy(tok_tab.at[idx], bufs[p], gsems[p])

    def wait_gather(p):
        pltpu.make_async_copy(tok_tab.at[pl.ds(0, CH)], bufs[p],
                              gsems[p]).wait()

    def issue_store(n, p):
        off = pl.multiple_of(base + n * CH, CH)
        pltpu.async_copy(bufs[p], out_hbm.at[pl.ds(off, CH)], osems[p])

    def wait_store(p):
        pltpu.make_async_copy(bufs[p], out_hbm.at[pl.ds(0, CH)],
                              osems[p]).wait()

    L = min(LOOKAHEAD, NCH)
    for n in range(L):
        issue_gather(n, n % NBUF)
    for n in range(NCH):
        wait_gather(n % NBUF)
        issue_store(n, n % NBUF)
        m = n + L
        if m < NCH:
            q = m % NBUF
            if m >= NBUF:
                wait_store(q)
            issue_gather(m, q)
    for k in range(max(0, NCH - NBUF), NCH):
        wait_store(k % NBUF)


def _sc_gather(ids_slice, token_table):
    mesh = plsc.VectorSubcoreMesh(
        core_axis_name="c", subcore_axis_name="s",
        num_cores=NC, num_subcores=NS)
    run = pl.kernel(
        _sc_body,
        out_type=jax.ShapeDtypeStruct((NTOKK, DMODEL), jnp.float32),
        mesh=mesh,
        compiler_params=pltpu.CompilerParams(needs_layout_passes=False),
        scratch_types=(
            [pltpu.VMEM((TOK_PER_W,), jnp.int32)]
            + [pltpu.VMEM((CH, DMODEL), jnp.float32)] * NBUF
            + [pltpu.SemaphoreType.DMA] * (2 * NBUF)
        ),
    )
    return run(ids_slice, token_table)


def _tc_body(x_ref, pos_ref, seg_ref, segtab_ref, lnw_ref,
             lnb_ref, out_ref):
    seg = seg_ref[...]
    lanes = lax.broadcasted_iota(jnp.int32, (BS, 3), 1)
    onehot = (seg == lanes).astype(jnp.float32)
    seg_emb = jnp.dot(onehot, segtab_ref[...],
                      preferred_element_type=jnp.float32)
    x = (x_ref[...] + pos_ref[...]) + seg_emb
    ones = jnp.ones((DMODEL, 1), jnp.float32)
    s1 = jnp.dot(x, ones, preferred_element_type=jnp.float32)
    s2 = jnp.dot(x * x, ones, preferred_element_type=jnp.float32)
    mean = s1 * (1.0 / DMODEL)
    var = s2 * (1.0 / DMODEL) - mean * mean
    rstd = lax.rsqrt(var + EPS)
    w = lnw_ref[...]
    out_ref[...] = (x * rstd - mean * rstd) * w + lnb_ref[...]


def _tc_finish(prev, x, segs_s, position_table, segment_table, lnw, lnb, s):
    blk_off = s * NBAT * SBLK

    def xmap(i):
        return ((i % NBAT) * SBLK + i // NBAT, 0)

    def omap(i):
        return (blk_off + (i % NBAT) * SBLK + i // NBAT, 0)

    def body(*refs):
        if prev is None:
            _tc_body(*refs)
        else:
            _tc_body(*refs[1:])

    in_specs = [
        pl.BlockSpec((BS, DMODEL), xmap),
        pl.BlockSpec((BS, DMODEL), lambda i: (i // NBAT, 0)),
        pl.BlockSpec((BS, 1), xmap),
        pl.BlockSpec((3, DMODEL), lambda i: (0, 0)),
        pl.BlockSpec((1, DMODEL), lambda i: (0, 0)),
        pl.BlockSpec((1, DMODEL), lambda i: (0, 0)),
    ]
    args = [x, position_table, segs_s, segment_table,
            lnw.reshape(1, DMODEL), lnb.reshape(1, DMODEL)]
    aliases = {}
    if prev is not None:
        in_specs = [pl.BlockSpec(memory_space=pl.ANY)] + in_specs
        args = [prev] + args
        aliases = {0: 0}
    return pl.pallas_call(
        body,
        grid=(GRIDK,),
        in_specs=in_specs,
        out_specs=pl.BlockSpec((BS, DMODEL), omap),
        out_shape=jax.ShapeDtypeStruct((NTOK, DMODEL), jnp.float32),
        input_output_aliases=aliases,
        compiler_params=pltpu.CompilerParams(
            dimension_semantics=("arbitrary",)),
    )(*args)


@jax.jit
def kernel(input_ids, segments, token_table, position_table, segment_table,
           ln_weight, ln_bias):
    ids = input_ids.reshape(NTOK).astype(jnp.int32)
    segs = segments.reshape(NTOK, 1).astype(jnp.int32)
    gathered = [
        _sc_gather(ids[s * NTOKK:(s + 1) * NTOKK], token_table)
        for s in range(NSLICE)
    ]
    out = None
    for s in range(NSLICE):
        out = _tc_finish(out, gathered[s],
                         segs[s * NTOKK:(s + 1) * NTOKK],
                         position_table, segment_table,
                         ln_weight, ln_bias, s)
    return out.reshape(BATCH, SEQLEN, DMODEL)
